# baseline (speedup 1.0000x reference)
import functools

import jax
import jax.numpy as jnp
from jax import lax
from jax.experimental import pallas as pl
from jax.experimental.pallas import tpu as pltpu
from jax.experimental.pallas import tpu_sc as plsc

VOCAB = 1000000
DIM = 64
BATCH = 16384
NBKT = VOCAB // 16

_INFO = plsc.get_sparse_core_info()
NC, NS = _INFO.num_cores, _INFO.num_subcores
NW = NC * NS
B_PER_W = BATCH // NW
NCHUNK = B_PER_W // 16


def _sc_gather_t(table2, idx):
    mesh = plsc.VectorSubcoreMesh(core_axis_name="c", subcore_axis_name="s")

    @functools.partial(
        pl.kernel,
        mesh=mesh,
        out_type=jax.ShapeDtypeStruct((DIM, BATCH), jnp.float32),
        scratch_types=[
            pltpu.VMEM((B_PER_W,), jnp.int32),
            pltpu.VMEM((B_PER_W,), jnp.int32),
            pltpu.VMEM((B_PER_W, 16), jnp.float32),
            pltpu.VMEM((B_PER_W,), jnp.float32),
            pltpu.SemaphoreType.DMA,
        ],
        compiler_params=pltpu.CompilerParams(
            use_tc_tiling_on_sc=False, needs_layout_passes=False),
    )
    def k(tbl_hbm, idx_hbm, out_hbm, ids_v, idx_v, buf_v, vals_v, sem):
        wid = lax.axis_index("s") * NC + lax.axis_index("c")
        base = wid * B_PER_W
        pltpu.sync_copy(idx_hbm.at[pl.ds(base, B_PER_W)], ids_v)

        def dloop(d, carry):
            off = d * NBKT

            def build(c, carry2):
                v = ids_v[pl.ds(c * 16, 16)]
                idx_v[pl.ds(c * 16, 16)] = (v >> 4) + off
                return carry2

            lax.fori_loop(0, NCHUNK, build, 0)
            pltpu.async_copy(tbl_hbm.at[idx_v], buf_v, sem).wait()

            def select(c, carry2):
                lanes = ids_v[pl.ds(c * 16, 16)] & 15
                rows = lax.iota(jnp.int32, 16) + c * 16
                vals_v[pl.ds(c * 16, 16)] = plsc.load_gather(
                    buf_v, [rows, lanes])
                return carry2

            lax.fori_loop(0, NCHUNK, select, 0)
            pltpu.sync_copy(vals_v, out_hbm.at[d, pl.ds(base, B_PER_W)])
            return carry

        lax.fori_loop(0, DIM, dloop, 0)

    return k(table2, idx)


def _tc_body(x_ref, w_ref, b_ref, o_ref):
    y = lax.dot_general(w_ref[...], x_ref[...], (((1,), (0,)), ((), ())),
                        preferred_element_type=jnp.float32)
    o_ref[...] = y + b_ref[...]


def _tc_linear_t(x_t, W, b2):
    blk = 2048
    return pl.pallas_call(
        _tc_body,
        grid=(BATCH // blk,),
        in_specs=[
            pl.BlockSpec((DIM, blk), lambda i: (0, i)),
            pl.BlockSpec((DIM, DIM), lambda i: (0, 0)),
            pl.BlockSpec((DIM, 1), lambda i: (0, 0)),
        ],
        out_specs=pl.BlockSpec((DIM, blk), lambda i: (0, i)),
        out_shape=jax.ShapeDtypeStruct((DIM, BATCH), jnp.float32),
    )(x_t, W, b2)


def kernel(input_doc_id, embedding_table, W, b):
    idx = input_doc_id.astype(jnp.int32)
    table2 = jnp.reshape(embedding_table.T, (DIM * NBKT, 16))
    x_t = _sc_gather_t(table2, idx)
    y_t = _tc_linear_t(x_t, W, b.reshape(DIM, 1))
    return y_t.T


# speedup vs baseline: 8.0577x; 8.0577x over previous
import functools

import jax
import jax.numpy as jnp
from jax import lax
from jax.experimental import pallas as pl
from jax.experimental.pallas import tpu as pltpu
from jax.experimental.pallas import tpu_sc as plsc

VOCAB = 1000000
DIM = 64
BATCH = 16384

_INFO = plsc.get_sparse_core_info()
NC, NS = _INFO.num_cores, _INFO.num_subcores
NW = NC * NS
B_PER_W = BATCH // NW


def _sc_gather(table, idx):
    mesh = plsc.VectorSubcoreMesh(core_axis_name="c", subcore_axis_name="s")

    @functools.partial(
        pl.kernel,
        mesh=mesh,
        out_type=jax.ShapeDtypeStruct((BATCH, DIM), jnp.float32),
        scratch_types=[
            pltpu.VMEM((B_PER_W,), jnp.int32),
            pltpu.VMEM((B_PER_W, DIM), jnp.float32),
            pltpu.SemaphoreType.DMA,
        ],
        compiler_params=pltpu.CompilerParams(use_tc_tiling_on_sc=False),
    )
    def k(tbl_hbm, idx_hbm, out_hbm, idx_v, rows_v, sem):
        wid = lax.axis_index("s") * NC + lax.axis_index("c")
        base = wid * B_PER_W
        pltpu.sync_copy(idx_hbm.at[pl.ds(base, B_PER_W)], idx_v)
        pltpu.async_copy(tbl_hbm.at[idx_v], rows_v, sem).wait()
        pltpu.sync_copy(rows_v, out_hbm.at[pl.ds(base, B_PER_W)])

    return k(table, idx)


def _tc_body(x_ref, w_ref, b_ref, o_ref):
    y = lax.dot_general(x_ref[...], w_ref[...], (((1,), (1,)), ((), ())),
                        preferred_element_type=jnp.float32)
    o_ref[...] = y + b_ref[...]


def _tc_linear(x, W, b2):
    blk = 2048
    return pl.pallas_call(
        _tc_body,
        grid=(BATCH // blk,),
        in_specs=[
            pl.BlockSpec((blk, DIM), lambda i: (i, 0)),
            pl.BlockSpec((DIM, DIM), lambda i: (0, 0)),
            pl.BlockSpec((1, DIM), lambda i: (0, 0)),
        ],
        out_specs=pl.BlockSpec((blk, DIM), lambda i: (i, 0)),
        out_shape=jax.ShapeDtypeStruct((BATCH, DIM), jnp.float32),
    )(x, W, b2)


def kernel(input_doc_id, embedding_table, W, b):
    idx = input_doc_id.astype(jnp.int32)
    rows = _sc_gather(embedding_table, idx)
    return _tc_linear(rows, W, b.reshape(1, DIM))


# speedup vs baseline: 8.1210x; 1.0079x over previous
import functools

import jax
import jax.numpy as jnp
from jax import lax
from jax.experimental import pallas as pl
from jax.experimental.pallas import tpu as pltpu
from jax.experimental.pallas import tpu_sc as plsc

VOCAB = 1000000
DIM = 64
BATCH = 16384

_INFO = plsc.get_sparse_core_info()
NC, NS = _INFO.num_cores, _INFO.num_subcores
NW = NC * NS
B_PER_W = BATCH // NW


def _sc_gather(table, idx):
    mesh = plsc.VectorSubcoreMesh(core_axis_name="c", subcore_axis_name="s")

    @functools.partial(
        pl.kernel,
        mesh=mesh,
        out_type=jax.ShapeDtypeStruct((BATCH, DIM), jnp.float32),
        scratch_types=[
            pltpu.VMEM((B_PER_W,), jnp.int32),
            pltpu.VMEM((B_PER_W, DIM), jnp.float32),
            pltpu.SemaphoreType.DMA,
        ],
        compiler_params=pltpu.CompilerParams(use_tc_tiling_on_sc=False),
    )
    def k(tbl_hbm, idx_hbm, out_hbm, idx_v, rows_v, sem):
        wid = lax.axis_index("s") * NC + lax.axis_index("c")
        base = wid * B_PER_W
        pltpu.sync_copy(idx_hbm.at[pl.ds(base, B_PER_W)], idx_v)
        pltpu.async_copy(tbl_hbm.at[idx_v], rows_v, sem).wait()
        pltpu.sync_copy(rows_v, out_hbm.at[pl.ds(base, B_PER_W)])

    return k(table, idx)


def _tc_body(x_ref, w_ref, b_ref, o_ref):
    y = lax.dot_general(w_ref[...], x_ref[...], (((1,), (1,)), ((), ())),
                        preferred_element_type=jnp.float32)
    o_ref[...] = y + b_ref[...]


def _tc_linear_t(x, W, b2):
    blk = 2048
    return pl.pallas_call(
        _tc_body,
        grid=(BATCH // blk,),
        in_specs=[
            pl.BlockSpec((blk, DIM), lambda i: (i, 0)),
            pl.BlockSpec((DIM, DIM), lambda i: (0, 0)),
            pl.BlockSpec((DIM, 1), lambda i: (0, 0)),
        ],
        out_specs=pl.BlockSpec((DIM, blk), lambda i: (0, i)),
        out_shape=jax.ShapeDtypeStruct((DIM, BATCH), jnp.float32),
    )(x, W, b2)


def kernel(input_doc_id, embedding_table, W, b):
    idx = input_doc_id.astype(jnp.int32)
    rows = _sc_gather(embedding_table, idx)
    return _tc_linear_t(rows, W, b.reshape(DIM, 1)).T
